# baseline (speedup 1.0000x reference)
import functools

import jax
import jax.numpy as jnp
from jax import lax
from jax.experimental import pallas as pl
from jax.experimental.pallas import tpu as pltpu
from jax.experimental.pallas import tpu_sc as plsc

NC = 2
NS = 16
LANES = 16


@functools.lru_cache(maxsize=None)
def _make_sc_scores(B, NEG, D, C):
    NW = NC * NS
    BPW = B // NW
    NCHUNK = BPW // C
    NIDX = C * NEG
    KROWS = NIDX // 128
    NV = D // LANES

    mesh = plsc.VectorSubcoreMesh(core_axis_name="c", subcore_axis_name="s")

    @functools.partial(
        pl.kernel,
        mesh=mesh,
        compiler_params=pltpu.CompilerParams(
            needs_layout_passes=False, use_tc_tiling_on_sc=False),
        out_type=[
            jax.ShapeDtypeStruct((B,), jnp.float32),
            jax.ShapeDtypeStruct((B * NEG,), jnp.float32),
        ],
        scratch_types=[
            pltpu.VMEM((C,), jnp.int32),
            pltpu.VMEM((C,), jnp.int32),
            pltpu.VMEM((NIDX,), jnp.int32),
            pltpu.VMEM((C, D), jnp.float32),
            pltpu.VMEM((C, D), jnp.float32),
            pltpu.VMEM((NIDX, D), jnp.float32),
            pltpu.VMEM((C,), jnp.float32),
            pltpu.VMEM((NIDX,), jnp.float32),
            pltpu.SemaphoreType.DMA,
        ],
    )
    def sc_scores(center_hbm, pos_hbm, negr_hbm, inw_hbm, outw_hbm,
                  pos_out, neg_out,
                  idx_c, idx_p, idx_n, v_rows, p_rows, n_rows,
                  pos_s, neg_s, sem):
        wid = lax.axis_index("s") * NC + lax.axis_index("c")
        base = wid * BPW

        def chunk(ci, chunk_carry):
            off = base + ci * C
            pltpu.sync_copy(center_hbm.at[pl.ds(off, C)], idx_c)
            pltpu.sync_copy(pos_hbm.at[pl.ds(off, C)], idx_p)
            pltpu.sync_copy(negr_hbm.at[pl.ds(off * NEG, NIDX)], idx_n)
            cps = [
                pltpu.async_copy(inw_hbm.at[idx_c], v_rows, sem),
                pltpu.async_copy(outw_hbm.at[idx_p], p_rows, sem),
            ]
            for j in range(KROWS):
                cps.append(pltpu.async_copy(
                    outw_hbm.at[idx_n.at[pl.ds(j * 128, 128)]],
                    n_rows.at[pl.ds(j * 128, 128)], sem))
            for cp in cps:
                cp.wait()

            lane = lax.iota(jnp.int32, LANES)

            def dot(vs, ref, r):
                acc = vs[0] * ref[r, pl.ds(0, 16)]
                for j in range(1, NV):
                    acc = acc + vs[j] * ref[r, pl.ds(16 * j, 16)]
                return jnp.sum(acc)

            def pos_group(g, carry):
                acc = jnp.zeros((LANES,), jnp.float32)
                for t in range(LANES):
                    i = g * LANES + t
                    vs = [v_rows[i, pl.ds(16 * j, 16)] for j in range(NV)]
                    s = dot(vs, p_rows, i)
                    acc = jnp.where(lane == t, s, acc)
                pos_s[pl.ds(g * LANES, LANES)] = acc
                return carry

            lax.fori_loop(0, C // LANES, pos_group, 0)

            def neg_group(g, carry):
                accs = [jnp.zeros((LANES,), jnp.float32) for _ in range(5)]
                for ai in range(4):
                    i = g * 4 + ai
                    vs = [v_rows[i, pl.ds(16 * j, 16)] for j in range(NV)]
                    for k in range(NEG):
                        rloc = ai * NEG + k
                        s = dot(vs, n_rows, i * NEG + k)
                        accs[rloc // LANES] = jnp.where(
                            lane == rloc % LANES, s, accs[rloc // LANES])
                for m in range(5):
                    neg_s[pl.ds(g * 4 * NEG + m * LANES, LANES)] = accs[m]
                return carry

            lax.fori_loop(0, C // 4, neg_group, 0)
            pltpu.sync_copy(pos_s, pos_out.at[pl.ds(off, C)])
            pltpu.sync_copy(neg_s, neg_out.at[pl.ds(off * NEG, NIDX)])
            return chunk_carry

        lax.fori_loop(0, NCHUNK, chunk, 0)

    return sc_scores


@functools.lru_cache(maxsize=None)
def _make_untile(V, D, CB):
    NW = NC * NS
    NFULL = (V // 128 * 128) // CB
    NTAIL = V - NFULL * CB
    mesh = plsc.VectorSubcoreMesh(core_axis_name="c", subcore_axis_name="s")

    @functools.partial(
        pl.kernel,
        mesh=mesh,
        compiler_params=pltpu.CompilerParams(
            needs_layout_passes=False, use_tc_tiling_on_sc=True),
        out_type=[
            jax.ShapeDtypeStruct((V * D,), jnp.float32),
            jax.ShapeDtypeStruct((V * D,), jnp.float32),
        ],
        scratch_types=[
            pltpu.VMEM((D, CB), jnp.float32),
            pltpu.VMEM((CB * D,), jnp.float32),
        ],
    )
    def untile(wt1_hbm, wt2_hbm, tail1_hbm, tail2_hbm, lin1_hbm, lin2_hbm,
               inb, outb):
        wid = lax.axis_index("s") * NC + lax.axis_index("c")

        for wt_hbm, lin_hbm in ((wt1_hbm, lin1_hbm), (wt2_hbm, lin2_hbm)):
            def block(i, carry, wt_hbm=wt_hbm, lin_hbm=lin_hbm):
                k = wid + NW * i

                @pl.when(k < NFULL)
                def _():
                    c0 = k * CB
                    pltpu.sync_copy(wt_hbm.at[:, pl.ds(c0, CB)], inb)

                    def row(p, rc):
                        for jj in range(D // 16):
                            g = plsc.load_gather(
                                inb, [lax.iota(jnp.int32, 16) + 16 * jj,
                                      jnp.full((16,), p, jnp.int32)])
                            outb[pl.ds(p * D + 16 * jj, 16)] = g
                        return rc

                    lax.fori_loop(0, CB, row, 0)
                    pltpu.sync_copy(outb, lin_hbm.at[pl.ds(c0 * D, CB * D)])
                return carry

            lax.fori_loop(0, (NFULL + NW - 1) // NW, block, 0)

        @pl.when(wid == 0)
        def _():
            pltpu.sync_copy(tail1_hbm,
                            lin1_hbm.at[pl.ds(NFULL * CB * D, NTAIL * D)])

        @pl.when(wid == 1)
        def _():
            pltpu.sync_copy(tail2_hbm,
                            lin2_hbm.at[pl.ds(NFULL * CB * D, NTAIL * D)])

    return untile


def _log_sigmoid(x):
    return jnp.minimum(x, 0.0) - jnp.log1p(jnp.exp(-jnp.abs(x)))


@functools.lru_cache(maxsize=None)
def _make_tc_loss(B, NEG):
    def body(pos_ref, neg_ref, out_ref):
        pos_ls = _log_sigmoid(pos_ref[...])
        neg_ls = _log_sigmoid(-neg_ref[...])
        out_ref[0, 0] = -(jnp.sum(pos_ls) + jnp.sum(neg_ls)) / B

    return pl.pallas_call(
        body,
        out_shape=jax.ShapeDtypeStruct((1, 1), jnp.float32),
        out_specs=pl.BlockSpec(memory_space=pltpu.SMEM),
    )


def kernel(center_words, pos_context, neg_context, in_weight, out_weight):
    B, NEG = neg_context.shape
    V, D = in_weight.shape
    cw = center_words.astype(jnp.int32)
    pc = pos_context.astype(jnp.int32)
    ncr = neg_context.astype(jnp.int32).reshape(B * NEG)
    cut = V // 128 * 128
    lin1, lin2 = _make_untile(V, D, 256)(
        in_weight.T, out_weight.T,
        in_weight[cut:, :].reshape(-1), out_weight[cut:, :].reshape(-1))
    pos_s, neg_s = _make_sc_scores(B, NEG, D, 32)(
        cw, pc, ncr, lin1.reshape(V, D), lin2.reshape(V, D))
    loss = _make_tc_loss(B, NEG)(
        pos_s.reshape(B // 128, 128), neg_s.reshape(B * NEG // 128, 128))
    return loss.reshape(())


# speedup vs baseline: 1.1512x; 1.1512x over previous
import functools

import jax
import jax.numpy as jnp
from jax import lax
from jax.experimental import pallas as pl
from jax.experimental.pallas import tpu as pltpu
from jax.experimental.pallas import tpu_sc as plsc

NC = 2
NS = 16
LANES = 16


@functools.lru_cache(maxsize=None)
def _make_sc_scores(B, NEG, D, C):
    NW = NC * NS
    BPW = B // NW
    NCHUNK = BPW // C
    NIDX = C * NEG
    KROWS = NIDX // 128
    NV = D // LANES

    mesh = plsc.VectorSubcoreMesh(core_axis_name="c", subcore_axis_name="s")

    @functools.partial(
        pl.kernel,
        mesh=mesh,
        compiler_params=pltpu.CompilerParams(
            needs_layout_passes=False, use_tc_tiling_on_sc=False),
        out_type=[
            jax.ShapeDtypeStruct((B,), jnp.float32),
            jax.ShapeDtypeStruct((B * NEG,), jnp.float32),
        ],
        scratch_types=[
            pltpu.VMEM((C,), jnp.int32),
            pltpu.VMEM((C,), jnp.int32),
            pltpu.VMEM((NIDX,), jnp.int32),
            pltpu.VMEM((C, D), jnp.float32),
            pltpu.VMEM((C, D), jnp.float32),
            pltpu.VMEM((NIDX, D), jnp.float32),
            pltpu.VMEM((C,), jnp.float32),
            pltpu.VMEM((NIDX,), jnp.float32),
            pltpu.SemaphoreType.DMA,
        ],
    )
    def sc_scores(center_hbm, pos_hbm, negr_hbm, inw_hbm, outw_hbm,
                  pos_out, neg_out,
                  idx_c, idx_p, idx_n, v_rows, p_rows, n_rows,
                  pos_s, neg_s, sem):
        wid = lax.axis_index("s") * NC + lax.axis_index("c")
        base = wid * BPW

        def chunk(ci, chunk_carry):
            off = base + ci * C
            pltpu.sync_copy(center_hbm.at[pl.ds(off, C)], idx_c)
            pltpu.sync_copy(pos_hbm.at[pl.ds(off, C)], idx_p)
            pltpu.sync_copy(negr_hbm.at[pl.ds(off * NEG, NIDX)], idx_n)
            cps = [
                pltpu.async_copy(inw_hbm.at[idx_c], v_rows, sem),
                pltpu.async_copy(outw_hbm.at[idx_p], p_rows, sem),
            ]
            for j in range(KROWS):
                cps.append(pltpu.async_copy(
                    outw_hbm.at[idx_n.at[pl.ds(j * 128, 128)]],
                    n_rows.at[pl.ds(j * 128, 128)], sem))
            for cp in cps:
                cp.wait()

            lane = lax.iota(jnp.int32, LANES)

            def dot(vs, ref, r):
                acc = vs[0] * ref[r, pl.ds(0, 16)]
                for j in range(1, NV):
                    acc = acc + vs[j] * ref[r, pl.ds(16 * j, 16)]
                return jnp.sum(acc)

            def pos_group(g, carry):
                acc = jnp.zeros((LANES,), jnp.float32)
                for t in range(LANES):
                    i = g * LANES + t
                    vs = [v_rows[i, pl.ds(16 * j, 16)] for j in range(NV)]
                    s = dot(vs, p_rows, i)
                    acc = jnp.where(lane == t, s, acc)
                pos_s[pl.ds(g * LANES, LANES)] = acc
                return carry

            lax.fori_loop(0, C // LANES, pos_group, 0)

            def neg_group(g, carry):
                accs = [jnp.zeros((LANES,), jnp.float32) for _ in range(5)]
                for ai in range(4):
                    i = g * 4 + ai
                    vs = [v_rows[i, pl.ds(16 * j, 16)] for j in range(NV)]
                    for k in range(NEG):
                        rloc = ai * NEG + k
                        s = dot(vs, n_rows, i * NEG + k)
                        accs[rloc // LANES] = jnp.where(
                            lane == rloc % LANES, s, accs[rloc // LANES])
                for m in range(5):
                    neg_s[pl.ds(g * 4 * NEG + m * LANES, LANES)] = accs[m]
                return carry

            lax.fori_loop(0, C // 4, neg_group, 0)
            pltpu.sync_copy(pos_s, pos_out.at[pl.ds(off, C)])
            pltpu.sync_copy(neg_s, neg_out.at[pl.ds(off * NEG, NIDX)])
            return chunk_carry

        lax.fori_loop(0, NCHUNK, chunk, 0)

    return sc_scores


@functools.lru_cache(maxsize=None)
def _make_untile(V, D, CB):
    NW = NC * NS
    NFULL = (V // 128 * 128) // CB
    NTAIL = V - NFULL * CB
    mesh = plsc.VectorSubcoreMesh(core_axis_name="c", subcore_axis_name="s")

    NPAIR = ((NFULL + NW - 1) // NW + 1) // 2

    @functools.partial(
        pl.kernel,
        mesh=mesh,
        compiler_params=pltpu.CompilerParams(
            needs_layout_passes=False, use_tc_tiling_on_sc=True),
        out_type=[
            jax.ShapeDtypeStruct((V * D,), jnp.float32),
            jax.ShapeDtypeStruct((V * D,), jnp.float32),
        ],
        scratch_types=[
            pltpu.VMEM((D, CB), jnp.float32),
            pltpu.VMEM((D, CB), jnp.float32),
            pltpu.VMEM((CB * D,), jnp.float32),
            pltpu.VMEM((CB * D,), jnp.float32),
            pltpu.SemaphoreType.DMA,
            pltpu.SemaphoreType.DMA,
            pltpu.SemaphoreType.DMA,
            pltpu.SemaphoreType.DMA,
        ],
    )
    def untile(wt1_hbm, wt2_hbm, tail1_hbm, tail2_hbm, lin1_hbm, lin2_hbm,
               inA, inB, outA, outB, siA, siB, soA, soB):
        wid = lax.axis_index("s") * NC + lax.axis_index("c")

        def transpose_block(inb, outb):
            def row(p, rc):
                for jj in range(D // 16):
                    g = plsc.load_gather(
                        inb, [lax.iota(jnp.int32, 16) + 16 * jj,
                              jnp.full((16,), p, jnp.int32)])
                    outb[pl.ds(p * D + 16 * jj, 16)] = g
                return rc

            lax.fori_loop(0, CB, row, 0, unroll=8)

        for wt_hbm, lin_hbm in ((wt1_hbm, lin1_hbm), (wt2_hbm, lin2_hbm)):
            def clamp(k):
                return jnp.minimum(k, NFULL - 1)

            def rd(k, buf, sem, wt_hbm=wt_hbm):
                return pltpu.async_copy(
                    wt_hbm.at[:, pl.ds(clamp(k) * CB, CB)], buf, sem)

            def wr(k, buf, sem, lin_hbm=lin_hbm):
                return pltpu.async_copy(
                    buf, lin_hbm.at[pl.ds(clamp(k) * CB * D, CB * D)], sem)

            b0 = wid
            b1 = wid + NW
            rd(b0, inA, siA).wait()
            cp = rd(b1, inB, siB)
            transpose_block(inA, outA)
            woA = wr(b0, outA, soA)
            cp.wait()
            rdnA = rd(wid + 2 * NW, inA, siA)
            transpose_block(inB, outB)
            woB = wr(b1, outB, soB)
            rdnB = rd(wid + 3 * NW, inB, siB)

            def pair(i2, carry, wt_hbm=wt_hbm, lin_hbm=lin_hbm):
                bA = wid + NW * 2 * i2
                bB = bA + NW
                pltpu.make_async_copy(
                    wt_hbm.at[:, pl.ds(0, CB)], inA, siA).wait()
                pltpu.make_async_copy(
                    outA, lin_hbm.at[pl.ds(0, CB * D)], soA).wait()
                transpose_block(inA, outA)
                wr(bA, outA, soA)
                rd(bA + 2 * NW, inA, siA)
                pltpu.make_async_copy(
                    wt_hbm.at[:, pl.ds(0, CB)], inB, siB).wait()
                pltpu.make_async_copy(
                    outB, lin_hbm.at[pl.ds(0, CB * D)], soB).wait()
                transpose_block(inB, outB)
                wr(bB, outB, soB)
                rd(bB + 2 * NW, inB, siB)
                return carry

            lax.fori_loop(1, NPAIR, pair, 0)
            pltpu.make_async_copy(wt_hbm.at[:, pl.ds(0, CB)], inA, siA).wait()
            pltpu.make_async_copy(wt_hbm.at[:, pl.ds(0, CB)], inB, siB).wait()
            pltpu.make_async_copy(outA, lin_hbm.at[pl.ds(0, CB * D)], soA).wait()
            pltpu.make_async_copy(outB, lin_hbm.at[pl.ds(0, CB * D)], soB).wait()

        @pl.when(wid == 0)
        def _():
            pltpu.sync_copy(tail1_hbm,
                            lin1_hbm.at[pl.ds(NFULL * CB * D, NTAIL * D)])

        @pl.when(wid == 1)
        def _():
            pltpu.sync_copy(tail2_hbm,
                            lin2_hbm.at[pl.ds(NFULL * CB * D, NTAIL * D)])

    return untile


def _log_sigmoid(x):
    return jnp.minimum(x, 0.0) - jnp.log1p(jnp.exp(-jnp.abs(x)))


@functools.lru_cache(maxsize=None)
def _make_tc_loss(B, NEG):
    def body(pos_ref, neg_ref, out_ref):
        pos_ls = _log_sigmoid(pos_ref[...])
        neg_ls = _log_sigmoid(-neg_ref[...])
        out_ref[0, 0] = -(jnp.sum(pos_ls) + jnp.sum(neg_ls)) / B

    return pl.pallas_call(
        body,
        out_shape=jax.ShapeDtypeStruct((1, 1), jnp.float32),
        out_specs=pl.BlockSpec(memory_space=pltpu.SMEM),
    )


def kernel(center_words, pos_context, neg_context, in_weight, out_weight):
    B, NEG = neg_context.shape
    V, D = in_weight.shape
    cw = center_words.astype(jnp.int32)
    pc = pos_context.astype(jnp.int32)
    ncr = neg_context.astype(jnp.int32).reshape(B * NEG)
    cut = V // 128 * 128
    lin1, lin2 = _make_untile(V, D, 256)(
        in_weight.T, out_weight.T,
        in_weight[cut:, :].reshape(-1), out_weight[cut:, :].reshape(-1))
    pos_s, neg_s = _make_sc_scores(B, NEG, D, 32)(
        cw, pc, ncr, lin1.reshape(V, D), lin2.reshape(V, D))
    loss = _make_tc_loss(B, NEG)(
        pos_s.reshape(B // 128, 128), neg_s.reshape(B * NEG // 128, 128))
    return loss.reshape(())


# speedup vs baseline: 3.0183x; 2.6219x over previous
import functools

import jax
import jax.numpy as jnp
from jax import lax
from jax.experimental import pallas as pl
from jax.experimental.pallas import tpu as pltpu
from jax.experimental.pallas import tpu_sc as plsc

NC = 2
NS = 16
LANES = 16


@functools.lru_cache(maxsize=None)
def _make_sc_scores(B, NEG, D, C):
    NW = NC * NS
    BPW = B // NW
    NCHUNK = BPW // C
    NIDX = C * NEG
    KROWS = NIDX // 128
    NV = D // LANES

    mesh = plsc.VectorSubcoreMesh(core_axis_name="c", subcore_axis_name="s")

    @functools.partial(
        pl.kernel,
        mesh=mesh,
        compiler_params=pltpu.CompilerParams(
            needs_layout_passes=False, use_tc_tiling_on_sc=False),
        out_type=[
            jax.ShapeDtypeStruct((B,), jnp.float32),
            jax.ShapeDtypeStruct((B * NEG,), jnp.float32),
        ],
        scratch_types=[
            pltpu.VMEM((C,), jnp.int32),
            pltpu.VMEM((C,), jnp.int32),
            pltpu.VMEM((NIDX,), jnp.int32),
            pltpu.VMEM((C, D), jnp.float32),
            pltpu.VMEM((C, D), jnp.float32),
            pltpu.VMEM((NIDX, D), jnp.float32),
            pltpu.VMEM((C,), jnp.float32),
            pltpu.VMEM((NIDX,), jnp.float32),
            pltpu.SemaphoreType.DMA,
        ],
    )
    def sc_scores(center_hbm, pos_hbm, negr_hbm, inw_hbm, outw_hbm,
                  pos_out, neg_out,
                  idx_c, idx_p, idx_n, v_rows, p_rows, n_rows,
                  pos_s, neg_s, sem):
        wid = lax.axis_index("s") * NC + lax.axis_index("c")
        base = wid * BPW

        def chunk(ci, chunk_carry):
            off = base + ci * C
            pltpu.sync_copy(center_hbm.at[pl.ds(off, C)], idx_c)
            pltpu.sync_copy(pos_hbm.at[pl.ds(off, C)], idx_p)
            pltpu.sync_copy(negr_hbm.at[pl.ds(off * NEG, NIDX)], idx_n)
            cps = [
                pltpu.async_copy(inw_hbm.at[idx_c], v_rows, sem),
                pltpu.async_copy(outw_hbm.at[idx_p], p_rows, sem),
            ]
            for j in range(KROWS):
                cps.append(pltpu.async_copy(
                    outw_hbm.at[idx_n.at[pl.ds(j * 128, 128)]],
                    n_rows.at[pl.ds(j * 128, 128)], sem))
            for cp in cps:
                cp.wait()

            lane = lax.iota(jnp.int32, LANES)

            def dot(vs, ref, r):
                acc = vs[0] * ref[r, pl.ds(0, 16)]
                for j in range(1, NV):
                    acc = acc + vs[j] * ref[r, pl.ds(16 * j, 16)]
                return jnp.sum(acc)

            def pos_group(g, carry):
                acc = jnp.zeros((LANES,), jnp.float32)
                for t in range(LANES):
                    i = g * LANES + t
                    vs = [v_rows[i, pl.ds(16 * j, 16)] for j in range(NV)]
                    s = dot(vs, p_rows, i)
                    acc = jnp.where(lane == t, s, acc)
                pos_s[pl.ds(g * LANES, LANES)] = acc
                return carry

            lax.fori_loop(0, C // LANES, pos_group, 0)

            def neg_group(g, carry):
                accs = [jnp.zeros((LANES,), jnp.float32) for _ in range(5)]
                for ai in range(4):
                    i = g * 4 + ai
                    vs = [v_rows[i, pl.ds(16 * j, 16)] for j in range(NV)]
                    for k in range(NEG):
                        rloc = ai * NEG + k
                        s = dot(vs, n_rows, i * NEG + k)
                        accs[rloc // LANES] = jnp.where(
                            lane == rloc % LANES, s, accs[rloc // LANES])
                for m in range(5):
                    neg_s[pl.ds(g * 4 * NEG + m * LANES, LANES)] = accs[m]
                return carry

            lax.fori_loop(0, C // 4, neg_group, 0)
            pltpu.sync_copy(pos_s, pos_out.at[pl.ds(off, C)])
            pltpu.sync_copy(neg_s, neg_out.at[pl.ds(off * NEG, NIDX)])
            return chunk_carry

        lax.fori_loop(0, NCHUNK, chunk, 0)

    return sc_scores


@functools.lru_cache(maxsize=None)
def _make_untile(V, D, CB):
    NW = NC * NS
    NFULL = (V // 128 * 128) // CB
    NTAIL = V - NFULL * CB
    mesh = plsc.VectorSubcoreMesh(core_axis_name="c", subcore_axis_name="s")

    NPAIR = ((NFULL + NW - 1) // NW + 1) // 2

    @functools.partial(
        pl.kernel,
        mesh=mesh,
        compiler_params=pltpu.CompilerParams(
            needs_layout_passes=False, use_tc_tiling_on_sc=True),
        out_type=[
            jax.ShapeDtypeStruct((V * D,), jnp.float32),
            jax.ShapeDtypeStruct((V * D,), jnp.float32),
        ],
        scratch_types=[
            pltpu.VMEM((D, CB), jnp.float32),
            pltpu.VMEM((D, CB), jnp.float32),
            pltpu.VMEM((CB * D,), jnp.float32),
            pltpu.VMEM((CB * D,), jnp.float32),
            pltpu.SemaphoreType.DMA,
            pltpu.SemaphoreType.DMA,
            pltpu.SemaphoreType.DMA,
            pltpu.SemaphoreType.DMA,
        ],
    )
    def untile(wt1_hbm, wt2_hbm, tail1_hbm, tail2_hbm, lin1_hbm, lin2_hbm,
               inA, inB, outA, outB, siA, siB, soA, soB):
        wid = lax.axis_index("s") * NC + lax.axis_index("c")

        def transpose_block(inb, outb):
            def row(p, rc):
                for jj in range(D // 16):
                    g = plsc.load_gather(
                        inb, [lax.iota(jnp.int32, 16) + 16 * jj,
                              jnp.full((16,), p, jnp.int32)])
                    outb[pl.ds(p * D + 16 * jj, 16)] = g
                return rc

            lax.fori_loop(0, CB, row, 0, unroll=8)

        for wt_hbm, lin_hbm in ((wt1_hbm, lin1_hbm), (wt2_hbm, lin2_hbm)):
            def clamp(k):
                return jnp.minimum(k, NFULL - 1)

            def rd(k, buf, sem, wt_hbm=wt_hbm):
                return pltpu.async_copy(
                    wt_hbm.at[:, pl.ds(clamp(k) * CB, CB)], buf, sem)

            def wr(k, buf, sem, lin_hbm=lin_hbm):
                return pltpu.async_copy(
                    buf, lin_hbm.at[pl.ds(clamp(k) * CB * D, CB * D)], sem)

            b0 = wid
            b1 = wid + NW
            rd(b0, inA, siA).wait()
            cp = rd(b1, inB, siB)
            transpose_block(inA, outA)
            woA = wr(b0, outA, soA)
            cp.wait()
            rdnA = rd(wid + 2 * NW, inA, siA)
            transpose_block(inB, outB)
            woB = wr(b1, outB, soB)
            rdnB = rd(wid + 3 * NW, inB, siB)

            def pair(i2, carry, wt_hbm=wt_hbm, lin_hbm=lin_hbm):
                bA = wid + NW * 2 * i2
                bB = bA + NW
                pltpu.make_async_copy(
                    wt_hbm.at[:, pl.ds(0, CB)], inA, siA).wait()
                pltpu.make_async_copy(
                    outA, lin_hbm.at[pl.ds(0, CB * D)], soA).wait()
                transpose_block(inA, outA)
                wr(bA, outA, soA)
                rd(bA + 2 * NW, inA, siA)
                pltpu.make_async_copy(
                    wt_hbm.at[:, pl.ds(0, CB)], inB, siB).wait()
                pltpu.make_async_copy(
                    outB, lin_hbm.at[pl.ds(0, CB * D)], soB).wait()
                transpose_block(inB, outB)
                wr(bB, outB, soB)
                rd(bB + 2 * NW, inB, siB)
                return carry

            lax.fori_loop(1, NPAIR, pair, 0)
            pltpu.make_async_copy(wt_hbm.at[:, pl.ds(0, CB)], inA, siA).wait()
            pltpu.make_async_copy(wt_hbm.at[:, pl.ds(0, CB)], inB, siB).wait()
            pltpu.make_async_copy(outA, lin_hbm.at[pl.ds(0, CB * D)], soA).wait()
            pltpu.make_async_copy(outB, lin_hbm.at[pl.ds(0, CB * D)], soB).wait()

        @pl.when(wid == 0)
        def _():
            pltpu.sync_copy(tail1_hbm,
                            lin1_hbm.at[pl.ds(NFULL * CB * D, NTAIL * D)])

        @pl.when(wid == 1)
        def _():
            pltpu.sync_copy(tail2_hbm,
                            lin2_hbm.at[pl.ds(NFULL * CB * D, NTAIL * D)])

    return untile


def _log_sigmoid(x):
    return jnp.minimum(x, 0.0) - jnp.log1p(jnp.exp(-jnp.abs(x)))


@functools.lru_cache(maxsize=None)
def _make_tc_loss(B, NEG):
    def body(pos_ref, neg_ref, out_ref):
        pos_ls = _log_sigmoid(pos_ref[...])
        neg_ls = _log_sigmoid(-neg_ref[...])
        out_ref[0, 0] = -(jnp.sum(pos_ls) + jnp.sum(neg_ls)) / B

    return pl.pallas_call(
        body,
        out_shape=jax.ShapeDtypeStruct((1, 1), jnp.float32),
        out_specs=pl.BlockSpec(memory_space=pltpu.SMEM),
    )


@functools.lru_cache(maxsize=None)
def _make_tc_relayout(V, D, CB):
    def body(x_ref, y_ref):
        t = x_ref[...].T
        t3 = t.reshape(CB // 2, 2, D)
        y_ref[...] = jnp.concatenate([t3[:, 0, :], t3[:, 1, :]], axis=-1)

    return pl.pallas_call(
        body,
        grid=((V + CB - 1) // CB,),
        in_specs=[pl.BlockSpec((D, CB), lambda i: (0, i))],
        out_specs=pl.BlockSpec((CB // 2, 2 * D), lambda i: (i, 0)),
        out_shape=jax.ShapeDtypeStruct((V // 2, 2 * D), jnp.float32),
    )


def kernel(center_words, pos_context, neg_context, in_weight, out_weight):
    B, NEG = neg_context.shape
    V, D = in_weight.shape
    cw = center_words.astype(jnp.int32)
    pc = pos_context.astype(jnp.int32)
    ncr = neg_context.astype(jnp.int32).reshape(B * NEG)
    relayout = _make_tc_relayout(V, D, 2048)
    lin1 = relayout(in_weight.T).reshape(V, D)
    lin2 = relayout(out_weight.T).reshape(V, D)
    pos_s, neg_s = _make_sc_scores(B, NEG, D, 32)(
        cw, pc, ncr, lin1, lin2)
    loss = _make_tc_loss(B, NEG)(
        pos_s.reshape(B // 128, 128), neg_s.reshape(B * NEG // 128, 128))
    return loss.reshape(())


# speedup vs baseline: 3.5257x; 1.1681x over previous
import functools

import jax
import jax.numpy as jnp
from jax import lax
from jax.experimental import pallas as pl
from jax.experimental.pallas import tpu as pltpu
from jax.experimental.pallas import tpu_sc as plsc

NC = 2
NS = 16
LANES = 16


@functools.lru_cache(maxsize=None)
def _make_sc_scores(B, NEG, D, C):
    NW = NC * NS
    BPW = B // NW
    NCHUNK = BPW // C
    NIDX = C * NEG
    KROWS = NIDX // 128
    NV = D // LANES

    mesh = plsc.VectorSubcoreMesh(core_axis_name="c", subcore_axis_name="s")

    @functools.partial(
        pl.kernel,
        mesh=mesh,
        compiler_params=pltpu.CompilerParams(
            needs_layout_passes=False, use_tc_tiling_on_sc=False),
        out_type=[
            jax.ShapeDtypeStruct((B,), jnp.float32),
            jax.ShapeDtypeStruct((B * NEG,), jnp.float32),
        ],
        scratch_types=[
            pltpu.VMEM((C,), jnp.int32),
            pltpu.VMEM((C,), jnp.int32),
            pltpu.VMEM((NIDX,), jnp.int32),
            pltpu.VMEM((C, D), jnp.float32),
            pltpu.VMEM((C, D), jnp.float32),
            pltpu.VMEM((NIDX, D), jnp.float32),
            pltpu.VMEM((C,), jnp.float32),
            pltpu.VMEM((NIDX,), jnp.float32),
            pltpu.SemaphoreType.DMA,
        ],
    )
    def sc_scores(center_hbm, pos_hbm, negr_hbm, inw_hbm, outw_hbm,
                  pos_out, neg_out,
                  idx_c, idx_p, idx_n, v_rows, p_rows, n_rows,
                  pos_s, neg_s, sem):
        wid = lax.axis_index("s") * NC + lax.axis_index("c")
        base = wid * BPW

        def chunk(ci, chunk_carry):
            off = base + ci * C
            pltpu.sync_copy(center_hbm.at[pl.ds(off, C)], idx_c)
            pltpu.sync_copy(pos_hbm.at[pl.ds(off, C)], idx_p)
            pltpu.sync_copy(negr_hbm.at[pl.ds(off * NEG, NIDX)], idx_n)
            cps = [
                pltpu.async_copy(inw_hbm.at[idx_c], v_rows, sem),
                pltpu.async_copy(outw_hbm.at[idx_p], p_rows, sem),
            ]
            for j in range(KROWS):
                cps.append(pltpu.async_copy(
                    outw_hbm.at[idx_n.at[pl.ds(j * 128, 128)]],
                    n_rows.at[pl.ds(j * 128, 128)], sem))
            for cp in cps:
                cp.wait()

            lane = lax.iota(jnp.int32, LANES)

            def dot(vs, ref, r):
                acc = vs[0] * ref[r, pl.ds(0, 16)]
                for j in range(1, NV):
                    acc = acc + vs[j] * ref[r, pl.ds(16 * j, 16)]
                return jnp.sum(acc)

            def pos_group(g, carry):
                acc = jnp.zeros((LANES,), jnp.float32)
                for t in range(LANES):
                    i = g * LANES + t
                    vs = [v_rows[i, pl.ds(16 * j, 16)] for j in range(NV)]
                    s = dot(vs, p_rows, i)
                    acc = jnp.where(lane == t, s, acc)
                pos_s[pl.ds(g * LANES, LANES)] = acc
                return carry

            lax.fori_loop(0, C // LANES, pos_group, 0)

            def neg_group(g, carry):
                accs = [jnp.zeros((LANES,), jnp.float32) for _ in range(5)]
                for ai in range(4):
                    i = g * 4 + ai
                    vs = [v_rows[i, pl.ds(16 * j, 16)] for j in range(NV)]
                    for k in range(NEG):
                        rloc = ai * NEG + k
                        s = dot(vs, n_rows, i * NEG + k)
                        accs[rloc // LANES] = jnp.where(
                            lane == rloc % LANES, s, accs[rloc // LANES])
                for m in range(5):
                    neg_s[pl.ds(g * 4 * NEG + m * LANES, LANES)] = accs[m]
                return carry

            lax.fori_loop(0, C // 4, neg_group, 0)
            pltpu.sync_copy(pos_s, pos_out.at[pl.ds(off, C)])
            pltpu.sync_copy(neg_s, neg_out.at[pl.ds(off * NEG, NIDX)])
            return chunk_carry

        lax.fori_loop(0, NCHUNK, chunk, 0)

    return sc_scores


@functools.lru_cache(maxsize=None)
def _make_untile(V, D, CB):
    NW = NC * NS
    NFULL = (V // 128 * 128) // CB
    NTAIL = V - NFULL * CB
    mesh = plsc.VectorSubcoreMesh(core_axis_name="c", subcore_axis_name="s")

    NPAIR = ((NFULL + NW - 1) // NW + 1) // 2

    @functools.partial(
        pl.kernel,
        mesh=mesh,
        compiler_params=pltpu.CompilerParams(
            needs_layout_passes=False, use_tc_tiling_on_sc=True),
        out_type=[
            jax.ShapeDtypeStruct((V * D,), jnp.float32),
            jax.ShapeDtypeStruct((V * D,), jnp.float32),
        ],
        scratch_types=[
            pltpu.VMEM((D, CB), jnp.float32),
            pltpu.VMEM((D, CB), jnp.float32),
            pltpu.VMEM((CB * D,), jnp.float32),
            pltpu.VMEM((CB * D,), jnp.float32),
            pltpu.SemaphoreType.DMA,
            pltpu.SemaphoreType.DMA,
            pltpu.SemaphoreType.DMA,
            pltpu.SemaphoreType.DMA,
        ],
    )
    def untile(wt1_hbm, wt2_hbm, tail1_hbm, tail2_hbm, lin1_hbm, lin2_hbm,
               inA, inB, outA, outB, siA, siB, soA, soB):
        wid = lax.axis_index("s") * NC + lax.axis_index("c")

        def transpose_block(inb, outb):
            def row(p, rc):
                for jj in range(D // 16):
                    g = plsc.load_gather(
                        inb, [lax.iota(jnp.int32, 16) + 16 * jj,
                              jnp.full((16,), p, jnp.int32)])
                    outb[pl.ds(p * D + 16 * jj, 16)] = g
                return rc

            lax.fori_loop(0, CB, row, 0, unroll=8)

        for wt_hbm, lin_hbm in ((wt1_hbm, lin1_hbm), (wt2_hbm, lin2_hbm)):
            def clamp(k):
                return jnp.minimum(k, NFULL - 1)

            def rd(k, buf, sem, wt_hbm=wt_hbm):
                return pltpu.async_copy(
                    wt_hbm.at[:, pl.ds(clamp(k) * CB, CB)], buf, sem)

            def wr(k, buf, sem, lin_hbm=lin_hbm):
                return pltpu.async_copy(
                    buf, lin_hbm.at[pl.ds(clamp(k) * CB * D, CB * D)], sem)

            b0 = wid
            b1 = wid + NW
            rd(b0, inA, siA).wait()
            cp = rd(b1, inB, siB)
            transpose_block(inA, outA)
            woA = wr(b0, outA, soA)
            cp.wait()
            rdnA = rd(wid + 2 * NW, inA, siA)
            transpose_block(inB, outB)
            woB = wr(b1, outB, soB)
            rdnB = rd(wid + 3 * NW, inB, siB)

            def pair(i2, carry, wt_hbm=wt_hbm, lin_hbm=lin_hbm):
                bA = wid + NW * 2 * i2
                bB = bA + NW
                pltpu.make_async_copy(
                    wt_hbm.at[:, pl.ds(0, CB)], inA, siA).wait()
                pltpu.make_async_copy(
                    outA, lin_hbm.at[pl.ds(0, CB * D)], soA).wait()
                transpose_block(inA, outA)
                wr(bA, outA, soA)
                rd(bA + 2 * NW, inA, siA)
                pltpu.make_async_copy(
                    wt_hbm.at[:, pl.ds(0, CB)], inB, siB).wait()
                pltpu.make_async_copy(
                    outB, lin_hbm.at[pl.ds(0, CB * D)], soB).wait()
                transpose_block(inB, outB)
                wr(bB, outB, soB)
                rd(bB + 2 * NW, inB, siB)
                return carry

            lax.fori_loop(1, NPAIR, pair, 0)
            pltpu.make_async_copy(wt_hbm.at[:, pl.ds(0, CB)], inA, siA).wait()
            pltpu.make_async_copy(wt_hbm.at[:, pl.ds(0, CB)], inB, siB).wait()
            pltpu.make_async_copy(outA, lin_hbm.at[pl.ds(0, CB * D)], soA).wait()
            pltpu.make_async_copy(outB, lin_hbm.at[pl.ds(0, CB * D)], soB).wait()

        @pl.when(wid == 0)
        def _():
            pltpu.sync_copy(tail1_hbm,
                            lin1_hbm.at[pl.ds(NFULL * CB * D, NTAIL * D)])

        @pl.when(wid == 1)
        def _():
            pltpu.sync_copy(tail2_hbm,
                            lin2_hbm.at[pl.ds(NFULL * CB * D, NTAIL * D)])

    return untile


def _log_sigmoid(x):
    return jnp.minimum(x, 0.0) - jnp.log1p(jnp.exp(-jnp.abs(x)))


@functools.lru_cache(maxsize=None)
def _make_tc_loss(B, NEG):
    def body(pos_ref, neg_ref, out_ref):
        pos_ls = _log_sigmoid(pos_ref[...])
        neg_ls = _log_sigmoid(-neg_ref[...])
        out_ref[0, 0] = -(jnp.sum(pos_ls) + jnp.sum(neg_ls)) / B

    return pl.pallas_call(
        body,
        out_shape=jax.ShapeDtypeStruct((1, 1), jnp.float32),
        out_specs=pl.BlockSpec(memory_space=pltpu.SMEM),
    )


@functools.lru_cache(maxsize=None)
def _make_tc_relayout(V, D, CB):
    NB = (V + CB - 1) // CB

    def body(x_ref, y_ref):
        t = x_ref[...].T
        y_ref[...] = jnp.concatenate([t[: CB // 2], t[CB // 2:]], axis=-1)

    return pl.pallas_call(
        body,
        grid=(NB,),
        in_specs=[pl.BlockSpec((D, CB), lambda i: (0, i))],
        out_specs=pl.BlockSpec((CB // 2, 2 * D), lambda i: (i, 0)),
        out_shape=jax.ShapeDtypeStruct((NB * CB // 2, 2 * D), jnp.float32),
    )


def kernel(center_words, pos_context, neg_context, in_weight, out_weight):
    B, NEG = neg_context.shape
    V, D = in_weight.shape
    CB = 2048
    NB = (V + CB - 1) // CB

    def remap(idx):
        idx = idx.astype(jnp.int32)
        blk = idx // CB
        rem = idx % CB
        return blk * CB + (rem % (CB // 2)) * 2 + rem // (CB // 2)

    cw = remap(center_words)
    pc = remap(pos_context)
    ncr = remap(neg_context).reshape(B * NEG)
    relayout = _make_tc_relayout(V, D, CB)
    lin1 = relayout(in_weight.T).reshape(NB * CB, D)
    lin2 = relayout(out_weight.T).reshape(NB * CB, D)
    pos_s, neg_s = _make_sc_scores(B, NEG, D, 32)(
        cw, pc, ncr, lin1, lin2)
    loss = _make_tc_loss(B, NEG)(
        pos_s.reshape(B // 128, 128), neg_s.reshape(B * NEG // 128, 128))
    return loss.reshape(())


# speedup vs baseline: 4.5961x; 1.3036x over previous
import functools

import jax
import jax.numpy as jnp
from jax import lax
from jax.experimental import pallas as pl
from jax.experimental.pallas import tpu as pltpu
from jax.experimental.pallas import tpu_sc as plsc

NC = 2
NS = 16
LANES = 16


@functools.lru_cache(maxsize=None)
def _make_sc_scores(B, NEG, D, C):
    NW = NC * NS
    BPW = B // NW
    NCHUNK = BPW // C
    NIDX = C * NEG
    KROWS = NIDX // 128
    NV = D // LANES

    mesh = plsc.VectorSubcoreMesh(core_axis_name="c", subcore_axis_name="s")

    @functools.partial(
        pl.kernel,
        mesh=mesh,
        compiler_params=pltpu.CompilerParams(
            needs_layout_passes=False, use_tc_tiling_on_sc=False),
        out_type=[
            jax.ShapeDtypeStruct((B,), jnp.float32),
            jax.ShapeDtypeStruct((B * NEG,), jnp.float32),
        ],
        scratch_types=[
            pltpu.VMEM((C,), jnp.int32),
            pltpu.VMEM((C,), jnp.int32),
            pltpu.VMEM((NIDX,), jnp.int32),
            pltpu.VMEM((C, D), jnp.float32),
            pltpu.VMEM((C, D), jnp.float32),
            pltpu.VMEM((NIDX, D), jnp.float32),
            pltpu.VMEM((C,), jnp.float32),
            pltpu.VMEM((NIDX,), jnp.float32),
            pltpu.SemaphoreType.DMA,
        ],
    )
    def sc_scores(center_hbm, pos_hbm, negr_hbm, inw_hbm, outw_hbm,
                  pos_out, neg_out,
                  idx_c, idx_p, idx_n, v_rows, p_rows, n_rows,
                  pos_s, neg_s, sem):
        wid = lax.axis_index("s") * NC + lax.axis_index("c")
        base = wid * BPW

        def chunk(ci, chunk_carry):
            off = base + ci * C
            pltpu.sync_copy(center_hbm.at[pl.ds(off, C)], idx_c)
            pltpu.sync_copy(pos_hbm.at[pl.ds(off, C)], idx_p)
            pltpu.sync_copy(negr_hbm.at[pl.ds(off * NEG, NIDX)], idx_n)
            cps = [
                pltpu.async_copy(inw_hbm.at[idx_c], v_rows, sem),
                pltpu.async_copy(outw_hbm.at[idx_p], p_rows, sem),
            ]
            for j in range(KROWS):
                cps.append(pltpu.async_copy(
                    outw_hbm.at[idx_n.at[pl.ds(j * 128, 128)]],
                    n_rows.at[pl.ds(j * 128, 128)], sem))
            for cp in cps:
                cp.wait()

            lane = lax.iota(jnp.int32, LANES)

            def dot(vs, ref, r):
                acc = vs[0] * ref[r, pl.ds(0, 16)]
                for j in range(1, NV):
                    acc = acc + vs[j] * ref[r, pl.ds(16 * j, 16)]
                return jnp.sum(acc)

            def pos_group(g, carry):
                acc = jnp.zeros((LANES,), jnp.float32)
                for t in range(LANES):
                    i = g * LANES + t
                    vs = [v_rows[i, pl.ds(16 * j, 16)] for j in range(NV)]
                    s = dot(vs, p_rows, i)
                    acc = jnp.where(lane == t, s, acc)
                pos_s[pl.ds(g * LANES, LANES)] = acc
                return carry

            lax.fori_loop(0, C // LANES, pos_group, 0)

            def neg_group(g, carry):
                accs = [jnp.zeros((LANES,), jnp.float32) for _ in range(5)]
                for ai in range(4):
                    i = g * 4 + ai
                    vs = [v_rows[i, pl.ds(16 * j, 16)] for j in range(NV)]
                    for k in range(NEG):
                        rloc = ai * NEG + k
                        s = dot(vs, n_rows, i * NEG + k)
                        accs[rloc // LANES] = jnp.where(
                            lane == rloc % LANES, s, accs[rloc // LANES])
                for m in range(5):
                    neg_s[pl.ds(g * 4 * NEG + m * LANES, LANES)] = accs[m]
                return carry

            lax.fori_loop(0, C // 4, neg_group, 0)
            pltpu.sync_copy(pos_s, pos_out.at[pl.ds(off, C)])
            pltpu.sync_copy(neg_s, neg_out.at[pl.ds(off * NEG, NIDX)])
            return chunk_carry

        lax.fori_loop(0, NCHUNK, chunk, 0)

    return sc_scores


@functools.lru_cache(maxsize=None)
def _make_untile(V, D, CB):
    NW = NC * NS
    NFULL = (V // 128 * 128) // CB
    NTAIL = V - NFULL * CB
    mesh = plsc.VectorSubcoreMesh(core_axis_name="c", subcore_axis_name="s")

    NPAIR = ((NFULL + NW - 1) // NW + 1) // 2

    @functools.partial(
        pl.kernel,
        mesh=mesh,
        compiler_params=pltpu.CompilerParams(
            needs_layout_passes=False, use_tc_tiling_on_sc=True),
        out_type=[
            jax.ShapeDtypeStruct((V * D,), jnp.float32),
            jax.ShapeDtypeStruct((V * D,), jnp.float32),
        ],
        scratch_types=[
            pltpu.VMEM((D, CB), jnp.float32),
            pltpu.VMEM((D, CB), jnp.float32),
            pltpu.VMEM((CB * D,), jnp.float32),
            pltpu.VMEM((CB * D,), jnp.float32),
            pltpu.SemaphoreType.DMA,
            pltpu.SemaphoreType.DMA,
            pltpu.SemaphoreType.DMA,
            pltpu.SemaphoreType.DMA,
        ],
    )
    def untile(wt1_hbm, wt2_hbm, tail1_hbm, tail2_hbm, lin1_hbm, lin2_hbm,
               inA, inB, outA, outB, siA, siB, soA, soB):
        wid = lax.axis_index("s") * NC + lax.axis_index("c")

        def transpose_block(inb, outb):
            def row(p, rc):
                for jj in range(D // 16):
                    g = plsc.load_gather(
                        inb, [lax.iota(jnp.int32, 16) + 16 * jj,
                              jnp.full((16,), p, jnp.int32)])
                    outb[pl.ds(p * D + 16 * jj, 16)] = g
                return rc

            lax.fori_loop(0, CB, row, 0, unroll=8)

        for wt_hbm, lin_hbm in ((wt1_hbm, lin1_hbm), (wt2_hbm, lin2_hbm)):
            def clamp(k):
                return jnp.minimum(k, NFULL - 1)

            def rd(k, buf, sem, wt_hbm=wt_hbm):
                return pltpu.async_copy(
                    wt_hbm.at[:, pl.ds(clamp(k) * CB, CB)], buf, sem)

            def wr(k, buf, sem, lin_hbm=lin_hbm):
                return pltpu.async_copy(
                    buf, lin_hbm.at[pl.ds(clamp(k) * CB * D, CB * D)], sem)

            b0 = wid
            b1 = wid + NW
            rd(b0, inA, siA).wait()
            cp = rd(b1, inB, siB)
            transpose_block(inA, outA)
            woA = wr(b0, outA, soA)
            cp.wait()
            rdnA = rd(wid + 2 * NW, inA, siA)
            transpose_block(inB, outB)
            woB = wr(b1, outB, soB)
            rdnB = rd(wid + 3 * NW, inB, siB)

            def pair(i2, carry, wt_hbm=wt_hbm, lin_hbm=lin_hbm):
                bA = wid + NW * 2 * i2
                bB = bA + NW
                pltpu.make_async_copy(
                    wt_hbm.at[:, pl.ds(0, CB)], inA, siA).wait()
                pltpu.make_async_copy(
                    outA, lin_hbm.at[pl.ds(0, CB * D)], soA).wait()
                transpose_block(inA, outA)
                wr(bA, outA, soA)
                rd(bA + 2 * NW, inA, siA)
                pltpu.make_async_copy(
                    wt_hbm.at[:, pl.ds(0, CB)], inB, siB).wait()
                pltpu.make_async_copy(
                    outB, lin_hbm.at[pl.ds(0, CB * D)], soB).wait()
                transpose_block(inB, outB)
                wr(bB, outB, soB)
                rd(bB + 2 * NW, inB, siB)
                return carry

            lax.fori_loop(1, NPAIR, pair, 0)
            pltpu.make_async_copy(wt_hbm.at[:, pl.ds(0, CB)], inA, siA).wait()
            pltpu.make_async_copy(wt_hbm.at[:, pl.ds(0, CB)], inB, siB).wait()
            pltpu.make_async_copy(outA, lin_hbm.at[pl.ds(0, CB * D)], soA).wait()
            pltpu.make_async_copy(outB, lin_hbm.at[pl.ds(0, CB * D)], soB).wait()

        @pl.when(wid == 0)
        def _():
            pltpu.sync_copy(tail1_hbm,
                            lin1_hbm.at[pl.ds(NFULL * CB * D, NTAIL * D)])

        @pl.when(wid == 1)
        def _():
            pltpu.sync_copy(tail2_hbm,
                            lin2_hbm.at[pl.ds(NFULL * CB * D, NTAIL * D)])

    return untile


def _log_sigmoid(x):
    return jnp.minimum(x, 0.0) - jnp.log1p(jnp.exp(-jnp.abs(x)))


@functools.lru_cache(maxsize=None)
def _make_tc_loss(B, NEG):
    def body(pos_ref, neg_ref, out_ref):
        pos_ls = _log_sigmoid(pos_ref[...])
        neg_ls = _log_sigmoid(-neg_ref[...])
        out_ref[0, 0] = -(jnp.sum(pos_ls) + jnp.sum(neg_ls)) / B

    return pl.pallas_call(
        body,
        out_shape=jax.ShapeDtypeStruct((1, 1), jnp.float32),
        out_specs=pl.BlockSpec(memory_space=pltpu.SMEM),
    )


@functools.lru_cache(maxsize=None)
def _make_tc_relayout(V, D, CB):
    NB = (V + CB - 1) // CB

    def body(x_ref, y_ref):
        t = x_ref[...].T
        y_ref[...] = jnp.concatenate([t[: CB // 2], t[CB // 2:]], axis=-1)

    return pl.pallas_call(
        body,
        grid=(NB,),
        in_specs=[pl.BlockSpec((D, CB), lambda i: (0, i))],
        out_specs=pl.BlockSpec((CB // 2, 2 * D), lambda i: (i, 0)),
        out_shape=jax.ShapeDtypeStruct((NB * CB // 2, 2 * D), jnp.float32),
    )


def kernel(center_words, pos_context, neg_context, in_weight, out_weight):
    B, NEG = neg_context.shape
    V, D = in_weight.shape
    CB = 4096
    NB = (V + CB - 1) // CB

    def remap(idx):
        idx = idx.astype(jnp.int32)
        blk = idx // CB
        rem = idx % CB
        return blk * CB + (rem % (CB // 2)) * 2 + rem // (CB // 2)

    cw = remap(center_words)
    pc = remap(pos_context)
    ncr = remap(neg_context).reshape(B * NEG)
    relayout = _make_tc_relayout(V, D, CB)
    lin1 = relayout(in_weight.T).reshape(NB * CB, D)
    lin2 = relayout(out_weight.T).reshape(NB * CB, D)
    pos_s, neg_s = _make_sc_scores(B, NEG, D, 32)(
        cw, pc, ncr, lin1, lin2)
    loss = _make_tc_loss(B, NEG)(
        pos_s.reshape(B // 128, 128), neg_s.reshape(B * NEG // 128, 128))
    return loss.reshape(())
